# baseline (speedup 1.0000x reference)
import functools

import jax
import jax.numpy as jnp
from jax import lax
from jax.experimental import pallas as pl
from jax.experimental.pallas import tpu as pltpu
from jax.experimental.pallas import tpu_sc as plsc

_N = 10000
_E = 160000
_D = 256
_H = _D // 2
_NC = 2
_NS = 16
_EPT = _E // _NS
_C = 80
_NCHUNK = _EPT // _C
_EPTP = _NCHUNK * _C
_NACC = _N + 16
_RPT = 624
_RTAIL = _N - _NS * _RPT
_BN = 1000


def _seg_sum_sc(yA, yB, src, dst, zrow, zdeg, with_deg):
    mesh = plsc.VectorSubcoreMesh(core_axis_name="c", subcore_axis_name="s")
    out_type = [jax.ShapeDtypeStruct((_NC, _N, _H), jnp.float32)]
    if with_deg:
        out_type.append(jax.ShapeDtypeStruct((_NACC,), jnp.float32))

    @functools.partial(
        pl.kernel,
        out_type=tuple(out_type),
        mesh=mesh,
        scratch_types=[
            pltpu.VMEM((_EPTP,), jnp.int32),
            pltpu.VMEM((_EPTP,), jnp.int32),
            pltpu.VMEM((_C,), jnp.int32),
            pltpu.VMEM((_C,), jnp.int32),
            pltpu.VMEM((_C, _H), jnp.float32),
            pltpu.VMEM((_C, _H), jnp.float32),
            pltpu.VMEM((_C,), jnp.float32),
            pltpu.VMEM_SHARED((_NACC, _H), jnp.float32),
            pltpu.VMEM_SHARED((_NACC,), jnp.float32),
            pltpu.SemaphoreType.DMA,
            pltpu.SemaphoreType.DMA,
        ],
    )
    def k(yA_hbm, yB_hbm, src_hbm, dst_hbm, zrow_hbm, zdeg_hbm,
          *out_and_scratch):
        if with_deg:
            s_out, deg_out = out_and_scratch[:2]
            scr = out_and_scratch[2:]
        else:
            s_out = out_and_scratch[0]
            scr = out_and_scratch[1:]
        (src_st, dst_st, dstA, dstB, bufA, bufB, ones_v,
         acc_sh, deg_sh, semA, semB) = scr
        c = lax.axis_index("c")
        s = lax.axis_index("s")
        if with_deg:
            for i in range(_C // 16):
                ones_v[pl.ds(i * 16, 16)] = jnp.ones((16,), jnp.float32)

        base = s * _EPTP
        pltpu.async_copy(src_hbm.at[pl.ds(base, _EPTP)], src_st, semA)
        pltpu.async_copy(dst_hbm.at[pl.ds(base, _EPTP)], dst_st, semB)

        @pl.when(s == 0)
        def _():
            pltpu.sync_copy(zrow_hbm, acc_sh)
            if with_deg:
                @pl.when(c == 0)
                def _():
                    pltpu.sync_copy(zdeg_hbm, deg_sh)

        pltpu.make_async_copy(src_hbm.at[pl.ds(0, _EPTP)], src_st, semA).wait()
        pltpu.make_async_copy(dst_hbm.at[pl.ds(0, _EPTP)], dst_st, semB).wait()
        plsc.subcore_barrier()

        def gather(kk, buf, sem):
            idx = src_st.at[pl.ds(kk * _C, _C)]

            @pl.when(c == 0)
            def _():
                pltpu.async_copy(yA_hbm.at[idx], buf, sem)

            @pl.when(c == 1)
            def _():
                pltpu.async_copy(yB_hbm.at[idx], buf, sem)

        def wait(buf, sem):
            pltpu.make_async_copy(
                yA_hbm.at[src_st.at[pl.ds(0, _C)]], buf, sem).wait()

        def load_dst(kk, dst_v):
            for i in range(_C // 16):
                dst_v[pl.ds(i * 16, 16)] = dst_st[pl.ds(kk * _C + i * 16, 16)]

        def scatter(kk, buf, dst_v):
            pltpu.sync_copy(buf, acc_sh.at[dst_v], add=True)
            if with_deg:
                @pl.when(c == 0)
                def _():
                    pltpu.sync_copy(ones_v, deg_sh.at[dst_v], add=True)

        gather(0, bufA, semA)

        @pl.loop(0, (_NCHUNK - 1) // 2)
        def _(i):
            k0 = 2 * i
            gather(k0 + 1, bufB, semB)
            load_dst(k0, dstA)
            wait(bufA, semA)
            scatter(k0, bufA, dstA)
            gather(k0 + 2, bufA, semA)
            load_dst(k0 + 1, dstB)
            wait(bufB, semB)
            scatter(k0 + 1, bufB, dstB)

        load_dst(_NCHUNK - 1, dstA)
        wait(bufA, semA)
        scatter(_NCHUNK - 1, bufA, dstA)

        plsc.subcore_barrier()
        pltpu.sync_copy(acc_sh.at[pl.ds(s * _RPT, _RPT)],
                        s_out.at[c, pl.ds(s * _RPT, _RPT)])

        @pl.when(s == _NS - 1)
        def _():
            pltpu.sync_copy(acc_sh.at[pl.ds(_NS * _RPT, _RTAIL)],
                            s_out.at[c, pl.ds(_NS * _RPT, _RTAIL)])

        if with_deg:
            @pl.when(jnp.logical_and(s == 0, c == 0))
            def _():
                pltpu.sync_copy(deg_sh, deg_out)

    return k(yA, yB, src, dst, zrow, zdeg)


def _mm(a, b):
    return jnp.dot(a, b, preferred_element_type=jnp.float32)


def _tc_y(x, Wl):
    def body(x_ref, wl_ref, yA_ref, yB_ref):
        y = _mm(x_ref[...], wl_ref[...])
        yA_ref[...] = y[:, :_H]
        yB_ref[...] = y[:, _H:]

    return pl.pallas_call(
        body,
        grid=(_N // _BN,),
        in_specs=[
            pl.BlockSpec((_BN, _D), lambda i: (i, 0)),
            pl.BlockSpec((_D, _D), lambda i: (0, 0)),
        ],
        out_specs=[
            pl.BlockSpec((_BN, _H), lambda i: (i, 0)),
            pl.BlockSpec((_BN, _H), lambda i: (i, 0)),
        ],
        out_shape=[
            jax.ShapeDtypeStruct((_N, _H), jnp.float32),
            jax.ShapeDtypeStruct((_N, _H), jnp.float32),
        ],
    )(x, Wl)


def _tc_z(x, Wr, b):
    def body(x_ref, wr_ref, b_ref, z_ref):
        z_ref[...] = _mm(x_ref[...], wr_ref[...]) + b_ref[...]

    return pl.pallas_call(
        body,
        grid=(_N // _BN,),
        in_specs=[
            pl.BlockSpec((_BN, _D), lambda i: (i, 0)),
            pl.BlockSpec((_D, _D), lambda i: (0, 0)),
            pl.BlockSpec((1, _D), lambda i: (0, 0)),
        ],
        out_specs=pl.BlockSpec((_BN, _D), lambda i: (i, 0)),
        out_shape=jax.ShapeDtypeStruct((_N, _D), jnp.float32),
    )(x, Wr, b.reshape(1, _D))


def _tc_mid(sA, sB, deg, z, Wl):
    def body(sA_ref, sB_ref, deg_ref, z_ref, wl_ref,
             yA_ref, yB_ref, hA_ref, hB_ref):
        rd = 1.0 / jnp.maximum(deg_ref[...], 1.0)
        zb = z_ref[...]
        hA = jnp.maximum(sA_ref[...] * rd + zb[:, :_H], 0.0)
        hB = jnp.maximum(sB_ref[...] * rd + zb[:, _H:], 0.0)
        hA_ref[...] = hA
        hB_ref[...] = hB
        wl = wl_ref[...]
        y2 = _mm(hA, wl[:_H, :]) + _mm(hB, wl[_H:, :])
        yA_ref[...] = y2[:, :_H]
        yB_ref[...] = y2[:, _H:]

    return pl.pallas_call(
        body,
        grid=(_N // _BN,),
        in_specs=[
            pl.BlockSpec((_BN, _H), lambda i: (i, 0)),
            pl.BlockSpec((_BN, _H), lambda i: (i, 0)),
            pl.BlockSpec((_BN, 1), lambda i: (i, 0)),
            pl.BlockSpec((_BN, _D), lambda i: (i, 0)),
            pl.BlockSpec((_D, _D), lambda i: (0, 0)),
        ],
        out_specs=[
            pl.BlockSpec((_BN, _H), lambda i: (i, 0)),
            pl.BlockSpec((_BN, _H), lambda i: (i, 0)),
            pl.BlockSpec((_BN, _H), lambda i: (i, 0)),
            pl.BlockSpec((_BN, _H), lambda i: (i, 0)),
        ],
        out_shape=[
            jax.ShapeDtypeStruct((_N, _H), jnp.float32),
            jax.ShapeDtypeStruct((_N, _H), jnp.float32),
            jax.ShapeDtypeStruct((_N, _H), jnp.float32),
            jax.ShapeDtypeStruct((_N, _H), jnp.float32),
        ],
    )(sA, sB, deg, z, Wl)


def _tc_z2(hA, hB, Wr, b):
    def body(hA_ref, hB_ref, wr_ref, b_ref, z_ref):
        wr = wr_ref[...]
        z_ref[...] = (_mm(hA_ref[...], wr[:_H, :]) +
                      _mm(hB_ref[...], wr[_H:, :]) + b_ref[...])

    return pl.pallas_call(
        body,
        grid=(_N // _BN,),
        in_specs=[
            pl.BlockSpec((_BN, _H), lambda i: (i, 0)),
            pl.BlockSpec((_BN, _H), lambda i: (i, 0)),
            pl.BlockSpec((_D, _D), lambda i: (0, 0)),
            pl.BlockSpec((1, _D), lambda i: (0, 0)),
        ],
        out_specs=pl.BlockSpec((_BN, _D), lambda i: (i, 0)),
        out_shape=jax.ShapeDtypeStruct((_N, _D), jnp.float32),
    )(hA, hB, Wr, b.reshape(1, _D))


def _tc_post(sA, sB, deg, z):
    def body(sA_ref, sB_ref, deg_ref, z_ref, o_ref):
        rd = 1.0 / jnp.maximum(deg_ref[...], 1.0)
        o_ref[...] = jnp.concatenate(
            [sA_ref[...] * rd, sB_ref[...] * rd], axis=1) + z_ref[...]

    return pl.pallas_call(
        body,
        grid=(_N // _BN,),
        in_specs=[
            pl.BlockSpec((_BN, _H), lambda i: (i, 0)),
            pl.BlockSpec((_BN, _H), lambda i: (i, 0)),
            pl.BlockSpec((_BN, 1), lambda i: (i, 0)),
            pl.BlockSpec((_BN, _D), lambda i: (i, 0)),
        ],
        out_specs=pl.BlockSpec((_BN, _D), lambda i: (i, 0)),
        out_shape=jax.ShapeDtypeStruct((_N, _D), jnp.float32),
    )(sA, sB, deg, z)


def kernel(x, edge_index, Wl1, Wr1, b1, Wl2, Wr2, b2):
    pad = _EPTP - _EPT
    src = edge_index[0].astype(jnp.int32).reshape(_NS, _EPT)
    dst = edge_index[1].astype(jnp.int32).reshape(_NS, _EPT)
    src = jnp.concatenate(
        [src, jnp.zeros((_NS, pad), jnp.int32)], axis=1).reshape(-1)
    dst = jnp.concatenate(
        [dst, jnp.full((_NS, pad), _N, jnp.int32)], axis=1).reshape(-1)
    zrow = jnp.zeros((_NACC, _H), jnp.float32)
    zdeg = jnp.zeros((_NACC,), jnp.float32)

    yA1, yB1 = _tc_y(x, Wl1)
    s1, deg1 = _seg_sum_sc(yA1, yB1, src, dst, zrow, zdeg, with_deg=True)
    z1 = _tc_z(x, Wr1, b1)
    deg = deg1[:_N].reshape(_N, 1)
    y2A, y2B, hA, hB = _tc_mid(s1[0], s1[1], deg, z1, Wl2)
    (s2,) = _seg_sum_sc(y2A, y2B, src, dst, zrow, zdeg, with_deg=False)
    z2 = _tc_z2(hA, hB, Wr2, b2)
    return _tc_post(s2[0], s2[1], deg, z2)


# speedup vs baseline: 1.1359x; 1.1359x over previous
import functools

import jax
import jax.numpy as jnp
from jax import lax
from jax.experimental import pallas as pl
from jax.experimental.pallas import tpu as pltpu
from jax.experimental.pallas import tpu_sc as plsc

_N = 10000
_E = 160000
_D = 256
_H = _D // 2
_NC = 2
_NS = 16
_EPT = _E // _NS
_C = 80
_NCHUNK = _EPT // _C
_EPTP = _NCHUNK * _C
_NACC = _N + 16
_RPT = 624
_RTAIL = _N - _NS * _RPT
_BN = 1000


def _seg_sum_sc(yA, yB, comb, zrow, zdeg, with_deg):
    mesh = plsc.VectorSubcoreMesh(core_axis_name="c", subcore_axis_name="s")
    out_type = [jax.ShapeDtypeStruct((_NC, _N, _H), jnp.float32)]
    if with_deg:
        out_type.append(jax.ShapeDtypeStruct((_NACC,), jnp.float32))

    @functools.partial(
        pl.kernel,
        out_type=tuple(out_type),
        mesh=mesh,
        scratch_types=[
            pltpu.VMEM((_EPTP,), jnp.int32),
            pltpu.VMEM((3, _C), jnp.int32),
            pltpu.VMEM((_C,), jnp.int32),
            pltpu.VMEM((_C,), jnp.int32),
            pltpu.VMEM((_C,), jnp.int32),
            pltpu.VMEM((_C, _H), jnp.float32),
            pltpu.VMEM((_C, _H), jnp.float32),
            pltpu.VMEM((_C, _H), jnp.float32),
            pltpu.VMEM((_C,), jnp.float32),
            pltpu.VMEM_SHARED((_NACC, _H), jnp.float32),
            pltpu.VMEM_SHARED((_NACC,), jnp.float32),
            pltpu.SemaphoreType.DMA,
            pltpu.SemaphoreType.DMA,
            pltpu.SemaphoreType.DMA,
            pltpu.SemaphoreType.DMA,
            pltpu.SemaphoreType.DMA,
            pltpu.SemaphoreType.DMA,
        ],
    )
    def k(yA_hbm, yB_hbm, comb_hbm, zrow_hbm, zdeg_hbm,
          *out_and_scratch):
        if with_deg:
            s_out, deg_out = out_and_scratch[:2]
            scr = out_and_scratch[2:]
        else:
            s_out = out_and_scratch[0]
            scr = out_and_scratch[1:]
        (comb_st, src_sl, dst0, dst1, dst2, buf0, buf1, buf2, ones_v,
         acc_sh, deg_sh, g0, g1, g2, s0, s1, s2) = scr
        dstV = (dst0, dst1, dst2)
        bufV = (buf0, buf1, buf2)
        gsem = (g0, g1, g2)
        ssem = (s0, s1, s2)
        c = lax.axis_index("c")
        s = lax.axis_index("s")
        if with_deg:
            for i in range(_C // 16):
                ones_v[pl.ds(i * 16, 16)] = jnp.ones((16,), jnp.float32)

        base = s * _EPTP
        pltpu.async_copy(comb_hbm.at[pl.ds(base, _EPTP)], comb_st, g0)

        @pl.when(s == 0)
        def _():
            pltpu.sync_copy(zrow_hbm, acc_sh)
            if with_deg:
                @pl.when(c == 0)
                def _():
                    pltpu.sync_copy(zdeg_hbm, deg_sh)

        pltpu.make_async_copy(comb_hbm.at[pl.ds(0, _EPTP)], comb_st, g0).wait()
        plsc.subcore_barrier()

        def unpack(kk, j):
            for i in range(_C // 16):
                v = comb_st[pl.ds(kk * _C + i * 16, 16)]
                src_sl[j, pl.ds(i * 16, 16)] = \
                    lax.shift_right_logical(v, 14)
                dstV[j][pl.ds(i * 16, 16)] = lax.bitwise_and(v, 16383)

        def gather(kk, j):
            del kk
            idx = src_sl.at[j]

            @pl.when(c == 0)
            def _():
                pltpu.async_copy(yA_hbm.at[idx], bufV[j], gsem[j])

            @pl.when(c == 1)
            def _():
                pltpu.async_copy(yB_hbm.at[idx], bufV[j], gsem[j])

        def wait_gather(j):
            pltpu.make_async_copy(
                yA_hbm.at[src_sl.at[j]], bufV[j], gsem[j]).wait()

        def ascatter(kk, j):
            del kk
            pltpu.async_copy(bufV[j], acc_sh.at[dstV[j]], ssem[j], add=True)
            if with_deg:
                @pl.when(c == 0)
                def _():
                    pltpu.async_copy(ones_v, deg_sh.at[dstV[j]], ssem[j],
                                     add=True)

        def wait_scatter(j):
            pltpu.make_async_copy(
                bufV[j], acc_sh.at[dstV[j]], ssem[j]).wait()
            if with_deg:
                @pl.when(c == 0)
                def _():
                    pltpu.make_async_copy(
                        ones_v, deg_sh.at[dstV[j]], ssem[j]).wait()

        unpack(0, 0)
        gather(0, 0)
        unpack(1, 1)
        gather(1, 1)

        def process(kk, j, j2):
            wait_gather(j)
            ascatter(kk, j)
            wait_scatter(j2)
            unpack(kk + 2, j2)
            gather(kk + 2, j2)

        wait_gather(0)
        ascatter(0, 0)
        unpack(2, 2)
        gather(2, 2)
        wait_gather(1)
        ascatter(1, 1)
        wait_scatter(0)
        unpack(3, 0)
        gather(3, 0)

        @pl.loop(0, (_NCHUNK - 5) // 3)
        def _(i):
            k0 = 3 * i + 2
            process(k0, 2, 1)
            process(k0 + 1, 0, 2)
            process(k0 + 2, 1, 0)

        process(_NCHUNK - 3, 2, 1)
        wait_gather(0)
        ascatter(_NCHUNK - 2, 0)
        wait_gather(1)
        ascatter(_NCHUNK - 1, 1)
        wait_scatter(2)
        wait_scatter(0)
        wait_scatter(1)

        plsc.subcore_barrier()
        pltpu.sync_copy(acc_sh.at[pl.ds(s * _RPT, _RPT)],
                        s_out.at[c, pl.ds(s * _RPT, _RPT)])

        @pl.when(s == _NS - 1)
        def _():
            pltpu.sync_copy(acc_sh.at[pl.ds(_NS * _RPT, _RTAIL)],
                            s_out.at[c, pl.ds(_NS * _RPT, _RTAIL)])

        if with_deg:
            @pl.when(jnp.logical_and(s == 0, c == 0))
            def _():
                pltpu.sync_copy(deg_sh, deg_out)

    return k(yA, yB, comb, zrow, zdeg)


def _mm(a, b):
    return jnp.dot(a, b, preferred_element_type=jnp.float32)


def _tc_y(x, Wl):
    def body(x_ref, wl_ref, yA_ref, yB_ref):
        y = _mm(x_ref[...], wl_ref[...])
        yA_ref[...] = y[:, :_H]
        yB_ref[...] = y[:, _H:]

    return pl.pallas_call(
        body,
        grid=(_N // _BN,),
        in_specs=[
            pl.BlockSpec((_BN, _D), lambda i: (i, 0)),
            pl.BlockSpec((_D, _D), lambda i: (0, 0)),
        ],
        out_specs=[
            pl.BlockSpec((_BN, _H), lambda i: (i, 0)),
            pl.BlockSpec((_BN, _H), lambda i: (i, 0)),
        ],
        out_shape=[
            jax.ShapeDtypeStruct((_N, _H), jnp.float32),
            jax.ShapeDtypeStruct((_N, _H), jnp.float32),
        ],
    )(x, Wl)


def _tc_z(x, Wr, b):
    def body(x_ref, wr_ref, b_ref, z_ref):
        z_ref[...] = _mm(x_ref[...], wr_ref[...]) + b_ref[...]

    return pl.pallas_call(
        body,
        grid=(_N // _BN,),
        in_specs=[
            pl.BlockSpec((_BN, _D), lambda i: (i, 0)),
            pl.BlockSpec((_D, _D), lambda i: (0, 0)),
            pl.BlockSpec((1, _D), lambda i: (0, 0)),
        ],
        out_specs=pl.BlockSpec((_BN, _D), lambda i: (i, 0)),
        out_shape=jax.ShapeDtypeStruct((_N, _D), jnp.float32),
    )(x, Wr, b.reshape(1, _D))


def _tc_mid(sA, sB, deg, z, Wl):
    def body(sA_ref, sB_ref, deg_ref, z_ref, wl_ref,
             yA_ref, yB_ref, hA_ref, hB_ref):
        rd = 1.0 / jnp.maximum(deg_ref[...], 1.0)
        zb = z_ref[...]
        hA = jnp.maximum(sA_ref[...] * rd + zb[:, :_H], 0.0)
        hB = jnp.maximum(sB_ref[...] * rd + zb[:, _H:], 0.0)
        hA_ref[...] = hA
        hB_ref[...] = hB
        wl = wl_ref[...]
        y2 = _mm(hA, wl[:_H, :]) + _mm(hB, wl[_H:, :])
        yA_ref[...] = y2[:, :_H]
        yB_ref[...] = y2[:, _H:]

    return pl.pallas_call(
        body,
        grid=(_N // _BN,),
        in_specs=[
            pl.BlockSpec((_BN, _H), lambda i: (i, 0)),
            pl.BlockSpec((_BN, _H), lambda i: (i, 0)),
            pl.BlockSpec((_BN, 1), lambda i: (i, 0)),
            pl.BlockSpec((_BN, _D), lambda i: (i, 0)),
            pl.BlockSpec((_D, _D), lambda i: (0, 0)),
        ],
        out_specs=[
            pl.BlockSpec((_BN, _H), lambda i: (i, 0)),
            pl.BlockSpec((_BN, _H), lambda i: (i, 0)),
            pl.BlockSpec((_BN, _H), lambda i: (i, 0)),
            pl.BlockSpec((_BN, _H), lambda i: (i, 0)),
        ],
        out_shape=[
            jax.ShapeDtypeStruct((_N, _H), jnp.float32),
            jax.ShapeDtypeStruct((_N, _H), jnp.float32),
            jax.ShapeDtypeStruct((_N, _H), jnp.float32),
            jax.ShapeDtypeStruct((_N, _H), jnp.float32),
        ],
    )(sA, sB, deg, z, Wl)


def _tc_z2(hA, hB, Wr, b):
    def body(hA_ref, hB_ref, wr_ref, b_ref, z_ref):
        wr = wr_ref[...]
        z_ref[...] = (_mm(hA_ref[...], wr[:_H, :]) +
                      _mm(hB_ref[...], wr[_H:, :]) + b_ref[...])

    return pl.pallas_call(
        body,
        grid=(_N // _BN,),
        in_specs=[
            pl.BlockSpec((_BN, _H), lambda i: (i, 0)),
            pl.BlockSpec((_BN, _H), lambda i: (i, 0)),
            pl.BlockSpec((_D, _D), lambda i: (0, 0)),
            pl.BlockSpec((1, _D), lambda i: (0, 0)),
        ],
        out_specs=pl.BlockSpec((_BN, _D), lambda i: (i, 0)),
        out_shape=jax.ShapeDtypeStruct((_N, _D), jnp.float32),
    )(hA, hB, Wr, b.reshape(1, _D))


def _tc_post(sA, sB, deg, z):
    def body(sA_ref, sB_ref, deg_ref, z_ref, o_ref):
        rd = 1.0 / jnp.maximum(deg_ref[...], 1.0)
        o_ref[...] = jnp.concatenate(
            [sA_ref[...] * rd, sB_ref[...] * rd], axis=1) + z_ref[...]

    return pl.pallas_call(
        body,
        grid=(_N // _BN,),
        in_specs=[
            pl.BlockSpec((_BN, _H), lambda i: (i, 0)),
            pl.BlockSpec((_BN, _H), lambda i: (i, 0)),
            pl.BlockSpec((_BN, 1), lambda i: (i, 0)),
            pl.BlockSpec((_BN, _D), lambda i: (i, 0)),
        ],
        out_specs=pl.BlockSpec((_BN, _D), lambda i: (i, 0)),
        out_shape=jax.ShapeDtypeStruct((_N, _D), jnp.float32),
    )(sA, sB, deg, z)


def kernel(x, edge_index, Wl1, Wr1, b1, Wl2, Wr2, b2):
    src = edge_index[0].astype(jnp.int32)
    dst = edge_index[1].astype(jnp.int32)
    comb = (src << 14) | dst
    zrow = jnp.zeros((_NACC, _H), jnp.float32)
    zdeg = jnp.zeros((_NACC,), jnp.float32)

    yA1, yB1 = _tc_y(x, Wl1)
    s1, deg1 = _seg_sum_sc(yA1, yB1, comb, zrow, zdeg, with_deg=True)
    z1 = _tc_z(x, Wr1, b1)
    deg = deg1[:_N].reshape(_N, 1)
    y2A, y2B, hA, hB = _tc_mid(s1[0], s1[1], deg, z1, Wl2)
    (s2,) = _seg_sum_sc(y2A, y2B, comb, zrow, zdeg, with_deg=False)
    z2 = _tc_z2(hA, hB, Wr2, b2)
    return _tc_post(s2[0], s2[1], deg, z2)


# speedup vs baseline: 1.1859x; 1.0440x over previous
import functools

import jax
import jax.numpy as jnp
from jax import lax
from jax.experimental import pallas as pl
from jax.experimental.pallas import tpu as pltpu
from jax.experimental.pallas import tpu_sc as plsc

_N = 10000
_E = 160000
_D = 256
_H = _D // 2
_NC = 2
_NS = 16
_EPT = _E // _NS
_C = 80
_NCHUNK = _EPT // _C
_EPTP = _NCHUNK * _C
_NACC = _N + 16
_RPT = 624
_RTAIL = _N - _NS * _RPT
_BN = 1000


def _seg_sum_sc(yA, yB, comb, zrow, zdeg, with_deg):
    mesh = plsc.VectorSubcoreMesh(core_axis_name="c", subcore_axis_name="s")
    out_type = [jax.ShapeDtypeStruct((_N, _H), jnp.float32),
                jax.ShapeDtypeStruct((_N, _H), jnp.float32)]
    if with_deg:
        out_type.append(jax.ShapeDtypeStruct((_NACC,), jnp.float32))

    @functools.partial(
        pl.kernel,
        out_type=tuple(out_type),
        mesh=mesh,
        scratch_types=[
            pltpu.VMEM((_EPTP,), jnp.int32),
            pltpu.VMEM((3, _C), jnp.int32),
            pltpu.VMEM((_C,), jnp.int32),
            pltpu.VMEM((_C,), jnp.int32),
            pltpu.VMEM((_C,), jnp.int32),
            pltpu.VMEM((_C, _H), jnp.float32),
            pltpu.VMEM((_C, _H), jnp.float32),
            pltpu.VMEM((_C, _H), jnp.float32),
            pltpu.VMEM((_C,), jnp.float32),
            pltpu.VMEM_SHARED((_NACC, _H), jnp.float32),
            pltpu.VMEM_SHARED((_NACC,), jnp.float32),
            pltpu.SemaphoreType.DMA,
            pltpu.SemaphoreType.DMA,
            pltpu.SemaphoreType.DMA,
            pltpu.SemaphoreType.DMA,
            pltpu.SemaphoreType.DMA,
            pltpu.SemaphoreType.DMA,
        ],
    )
    def k(yA_hbm, yB_hbm, comb_hbm, zrow_hbm, zdeg_hbm,
          *out_and_scratch):
        if with_deg:
            sA_out, sB_out, deg_out = out_and_scratch[:3]
            scr = out_and_scratch[3:]
        else:
            sA_out, sB_out = out_and_scratch[:2]
            scr = out_and_scratch[2:]
        (comb_st, src_sl, dst0, dst1, dst2, buf0, buf1, buf2, ones_v,
         acc_sh, deg_sh, g0, g1, g2, s0, s1, s2) = scr
        dstV = (dst0, dst1, dst2)
        bufV = (buf0, buf1, buf2)
        gsem = (g0, g1, g2)
        ssem = (s0, s1, s2)
        c = lax.axis_index("c")
        s = lax.axis_index("s")
        if with_deg:
            for i in range(_C // 16):
                ones_v[pl.ds(i * 16, 16)] = jnp.ones((16,), jnp.float32)

        base = s * _EPTP
        pltpu.async_copy(comb_hbm.at[pl.ds(base, _EPTP)], comb_st, g0)

        @pl.when(s == 0)
        def _():
            pltpu.sync_copy(zrow_hbm, acc_sh)
            if with_deg:
                @pl.when(c == 0)
                def _():
                    pltpu.sync_copy(zdeg_hbm, deg_sh)

        pltpu.make_async_copy(comb_hbm.at[pl.ds(0, _EPTP)], comb_st, g0).wait()
        plsc.subcore_barrier()

        def unpack(kk, j):
            for i in range(_C // 16):
                v = comb_st[pl.ds(kk * _C + i * 16, 16)]
                src_sl[j, pl.ds(i * 16, 16)] = \
                    lax.shift_right_logical(v, 14)
                dstV[j][pl.ds(i * 16, 16)] = lax.bitwise_and(v, 16383)

        def gather(kk, j):
            del kk
            idx = src_sl.at[j]

            @pl.when(c == 0)
            def _():
                pltpu.async_copy(yA_hbm.at[idx], bufV[j], gsem[j])

            @pl.when(c == 1)
            def _():
                pltpu.async_copy(yB_hbm.at[idx], bufV[j], gsem[j])

        def wait_gather(j):
            pltpu.make_async_copy(
                yA_hbm.at[src_sl.at[j]], bufV[j], gsem[j]).wait()

        def ascatter(kk, j):
            del kk
            pltpu.async_copy(bufV[j], acc_sh.at[dstV[j]], ssem[j], add=True)
            if with_deg:
                @pl.when(c == 0)
                def _():
                    pltpu.async_copy(ones_v, deg_sh.at[dstV[j]], ssem[j],
                                     add=True)

        def wait_scatter(j):
            pltpu.make_async_copy(
                bufV[j], acc_sh.at[dstV[j]], ssem[j]).wait()
            if with_deg:
                @pl.when(c == 0)
                def _():
                    pltpu.make_async_copy(
                        ones_v, deg_sh.at[dstV[j]], ssem[j]).wait()

        unpack(0, 0)
        gather(0, 0)
        unpack(1, 1)
        gather(1, 1)

        def process(kk, j, j2):
            wait_gather(j)
            ascatter(kk, j)
            wait_scatter(j2)
            unpack(kk + 2, j2)
            gather(kk + 2, j2)

        wait_gather(0)
        ascatter(0, 0)
        unpack(2, 2)
        gather(2, 2)
        wait_gather(1)
        ascatter(1, 1)
        wait_scatter(0)
        unpack(3, 0)
        gather(3, 0)

        @pl.loop(0, (_NCHUNK - 5) // 3)
        def _(i):
            k0 = 3 * i + 2
            process(k0, 2, 1)
            process(k0 + 1, 0, 2)
            process(k0 + 2, 1, 0)

        process(_NCHUNK - 3, 2, 1)
        wait_gather(0)
        ascatter(_NCHUNK - 2, 0)
        wait_gather(1)
        ascatter(_NCHUNK - 1, 1)
        wait_scatter(2)
        wait_scatter(0)
        wait_scatter(1)

        plsc.subcore_barrier()
        @pl.when(c == 0)
        def _():
            pltpu.sync_copy(acc_sh.at[pl.ds(s * _RPT, _RPT)],
                            sA_out.at[pl.ds(s * _RPT, _RPT)])

            @pl.when(s == _NS - 1)
            def _():
                pltpu.sync_copy(acc_sh.at[pl.ds(_NS * _RPT, _RTAIL)],
                                sA_out.at[pl.ds(_NS * _RPT, _RTAIL)])

        @pl.when(c == 1)
        def _():
            pltpu.sync_copy(acc_sh.at[pl.ds(s * _RPT, _RPT)],
                            sB_out.at[pl.ds(s * _RPT, _RPT)])

            @pl.when(s == _NS - 1)
            def _():
                pltpu.sync_copy(acc_sh.at[pl.ds(_NS * _RPT, _RTAIL)],
                                sB_out.at[pl.ds(_NS * _RPT, _RTAIL)])

        if with_deg:
            @pl.when(jnp.logical_and(s == 0, c == 0))
            def _():
                pltpu.sync_copy(deg_sh, deg_out)

    return k(yA, yB, comb, zrow, zdeg)


def _mm(a, b):
    return jnp.dot(a, b, preferred_element_type=jnp.float32)


def _tc_y(x, Wl):
    def body(x_ref, wl_ref, yA_ref, yB_ref):
        y = _mm(x_ref[...], wl_ref[...])
        yA_ref[...] = y[:, :_H]
        yB_ref[...] = y[:, _H:]

    return pl.pallas_call(
        body,
        grid=(_N // _BN,),
        in_specs=[
            pl.BlockSpec((_BN, _D), lambda i: (i, 0)),
            pl.BlockSpec((_D, _D), lambda i: (0, 0)),
        ],
        out_specs=[
            pl.BlockSpec((_BN, _H), lambda i: (i, 0)),
            pl.BlockSpec((_BN, _H), lambda i: (i, 0)),
        ],
        out_shape=[
            jax.ShapeDtypeStruct((_N, _H), jnp.float32),
            jax.ShapeDtypeStruct((_N, _H), jnp.float32),
        ],
    )(x, Wl)


def _tc_z(x, Wr, b):
    def body(x_ref, wr_ref, b_ref, z_ref):
        z_ref[...] = _mm(x_ref[...], wr_ref[...]) + b_ref[...]

    return pl.pallas_call(
        body,
        grid=(_N // _BN,),
        in_specs=[
            pl.BlockSpec((_BN, _D), lambda i: (i, 0)),
            pl.BlockSpec((_D, _D), lambda i: (0, 0)),
            pl.BlockSpec((1, _D), lambda i: (0, 0)),
        ],
        out_specs=pl.BlockSpec((_BN, _D), lambda i: (i, 0)),
        out_shape=jax.ShapeDtypeStruct((_N, _D), jnp.float32),
    )(x, Wr, b.reshape(1, _D))


def _tc_mid(sA, sB, deg, z, Wl):
    def body(sA_ref, sB_ref, deg_ref, z_ref, wl_ref,
             yA_ref, yB_ref, hA_ref, hB_ref):
        rd = 1.0 / jnp.maximum(deg_ref[...], 1.0)
        zb = z_ref[...]
        hA = jnp.maximum(sA_ref[...] * rd + zb[:, :_H], 0.0)
        hB = jnp.maximum(sB_ref[...] * rd + zb[:, _H:], 0.0)
        hA_ref[...] = hA
        hB_ref[...] = hB
        wl = wl_ref[...]
        y2 = _mm(hA, wl[:_H, :]) + _mm(hB, wl[_H:, :])
        yA_ref[...] = y2[:, :_H]
        yB_ref[...] = y2[:, _H:]

    return pl.pallas_call(
        body,
        grid=(_N // _BN,),
        in_specs=[
            pl.BlockSpec((_BN, _H), lambda i: (i, 0)),
            pl.BlockSpec((_BN, _H), lambda i: (i, 0)),
            pl.BlockSpec((_BN, 1), lambda i: (i, 0)),
            pl.BlockSpec((_BN, _D), lambda i: (i, 0)),
            pl.BlockSpec((_D, _D), lambda i: (0, 0)),
        ],
        out_specs=[
            pl.BlockSpec((_BN, _H), lambda i: (i, 0)),
            pl.BlockSpec((_BN, _H), lambda i: (i, 0)),
            pl.BlockSpec((_BN, _H), lambda i: (i, 0)),
            pl.BlockSpec((_BN, _H), lambda i: (i, 0)),
        ],
        out_shape=[
            jax.ShapeDtypeStruct((_N, _H), jnp.float32),
            jax.ShapeDtypeStruct((_N, _H), jnp.float32),
            jax.ShapeDtypeStruct((_N, _H), jnp.float32),
            jax.ShapeDtypeStruct((_N, _H), jnp.float32),
        ],
    )(sA, sB, deg, z, Wl)


def _tc_z2(hA, hB, Wr, b):
    def body(hA_ref, hB_ref, wr_ref, b_ref, z_ref):
        wr = wr_ref[...]
        z_ref[...] = (_mm(hA_ref[...], wr[:_H, :]) +
                      _mm(hB_ref[...], wr[_H:, :]) + b_ref[...])

    return pl.pallas_call(
        body,
        grid=(_N // _BN,),
        in_specs=[
            pl.BlockSpec((_BN, _H), lambda i: (i, 0)),
            pl.BlockSpec((_BN, _H), lambda i: (i, 0)),
            pl.BlockSpec((_D, _D), lambda i: (0, 0)),
            pl.BlockSpec((1, _D), lambda i: (0, 0)),
        ],
        out_specs=pl.BlockSpec((_BN, _D), lambda i: (i, 0)),
        out_shape=jax.ShapeDtypeStruct((_N, _D), jnp.float32),
    )(hA, hB, Wr, b.reshape(1, _D))


def _tc_post(sA, sB, deg, z):
    def body(sA_ref, sB_ref, deg_ref, z_ref, o_ref):
        rd = 1.0 / jnp.maximum(deg_ref[...], 1.0)
        o_ref[...] = jnp.concatenate(
            [sA_ref[...] * rd, sB_ref[...] * rd], axis=1) + z_ref[...]

    return pl.pallas_call(
        body,
        grid=(_N // _BN,),
        in_specs=[
            pl.BlockSpec((_BN, _H), lambda i: (i, 0)),
            pl.BlockSpec((_BN, _H), lambda i: (i, 0)),
            pl.BlockSpec((_BN, 1), lambda i: (i, 0)),
            pl.BlockSpec((_BN, _D), lambda i: (i, 0)),
        ],
        out_specs=pl.BlockSpec((_BN, _D), lambda i: (i, 0)),
        out_shape=jax.ShapeDtypeStruct((_N, _D), jnp.float32),
    )(sA, sB, deg, z)


def kernel(x, edge_index, Wl1, Wr1, b1, Wl2, Wr2, b2):
    src = edge_index[0].astype(jnp.int32)
    dst = edge_index[1].astype(jnp.int32)
    comb = (src << 14) | dst
    zrow = jnp.zeros((_NACC, _H), jnp.float32)
    zdeg = jnp.zeros((_NACC,), jnp.float32)

    yA1, yB1 = _tc_y(x, Wl1)
    s1A, s1B, deg1 = _seg_sum_sc(yA1, yB1, comb, zrow, zdeg, with_deg=True)
    z1 = _tc_z(x, Wr1, b1)
    deg = deg1[:_N].reshape(_N, 1)
    y2A, y2B, hA, hB = _tc_mid(s1A, s1B, deg, z1, Wl2)
    s2A, s2B = _seg_sum_sc(y2A, y2B, comb, zrow, zdeg, with_deg=False)
    z2 = _tc_z2(hA, hB, Wr2, b2)
    return _tc_post(s2A, s2B, deg, z2)


# speedup vs baseline: 1.1874x; 1.0012x over previous
import functools

import jax
import jax.numpy as jnp
from jax import lax
from jax.experimental import pallas as pl
from jax.experimental.pallas import tpu as pltpu
from jax.experimental.pallas import tpu_sc as plsc

_N = 10000
_E = 160000
_D = 256
_H = _D // 2
_NC = 2
_NS = 16
_EPT = _E // _NS
_C = 80
_NCHUNK = _EPT // _C
_EPTP = _NCHUNK * _C
_NACC = _N + 16
_RPT = 624
_RTAIL = _N - _NS * _RPT
_ZTAIL = _NACC - _NS * _RPT
_BN = 1000


def _seg_sum_sc(yA, yB, comb, zrow, zdeg, with_deg):
    mesh = plsc.VectorSubcoreMesh(core_axis_name="c", subcore_axis_name="s")
    out_type = [jax.ShapeDtypeStruct((_N, _H), jnp.float32),
                jax.ShapeDtypeStruct((_N, _H), jnp.float32)]
    if with_deg:
        out_type.append(jax.ShapeDtypeStruct((_NACC,), jnp.float32))

    @functools.partial(
        pl.kernel,
        out_type=tuple(out_type),
        mesh=mesh,
        scratch_types=[
            pltpu.VMEM((_EPTP,), jnp.int32),
            pltpu.VMEM((3, _C), jnp.int32),
            pltpu.VMEM((_C,), jnp.int32),
            pltpu.VMEM((_C,), jnp.int32),
            pltpu.VMEM((_C,), jnp.int32),
            pltpu.VMEM((_C, _H), jnp.float32),
            pltpu.VMEM((_C, _H), jnp.float32),
            pltpu.VMEM((_C, _H), jnp.float32),
            pltpu.VMEM((_C,), jnp.float32),
            pltpu.VMEM_SHARED((_NACC, _H), jnp.float32),
            pltpu.VMEM_SHARED((_NACC,), jnp.float32),
            pltpu.SemaphoreType.DMA,
            pltpu.SemaphoreType.DMA,
            pltpu.SemaphoreType.DMA,
            pltpu.SemaphoreType.DMA,
            pltpu.SemaphoreType.DMA,
            pltpu.SemaphoreType.DMA,
        ],
    )
    def k(yA_hbm, yB_hbm, comb_hbm, zrow_hbm, zdeg_hbm,
          *out_and_scratch):
        if with_deg:
            sA_out, sB_out, deg_out = out_and_scratch[:3]
            scr = out_and_scratch[3:]
        else:
            sA_out, sB_out = out_and_scratch[:2]
            scr = out_and_scratch[2:]
        (comb_st, src_sl, dst0, dst1, dst2, buf0, buf1, buf2, ones_v,
         acc_sh, deg_sh, g0, g1, g2, s0, s1, s2) = scr
        dstV = (dst0, dst1, dst2)
        bufV = (buf0, buf1, buf2)
        gsem = (g0, g1, g2)
        ssem = (s0, s1, s2)
        c = lax.axis_index("c")
        s = lax.axis_index("s")
        if with_deg:
            for i in range(_C // 16):
                ones_v[pl.ds(i * 16, 16)] = jnp.ones((16,), jnp.float32)

        base = s * _EPTP
        pltpu.async_copy(comb_hbm.at[pl.ds(base, _EPTP)], comb_st, g0)

        zlo = s * _RPT
        pltpu.sync_copy(zrow_hbm.at[pl.ds(zlo, _RPT)],
                        acc_sh.at[pl.ds(zlo, _RPT)])

        @pl.when(s == _NS - 1)
        def _():
            pltpu.sync_copy(zrow_hbm.at[pl.ds(_NS * _RPT, _ZTAIL)],
                            acc_sh.at[pl.ds(_NS * _RPT, _ZTAIL)])

        if with_deg:
            @pl.when(jnp.logical_and(s == 0, c == 0))
            def _():
                pltpu.sync_copy(zdeg_hbm, deg_sh)

        pltpu.make_async_copy(comb_hbm.at[pl.ds(0, _EPTP)], comb_st, g0).wait()
        plsc.subcore_barrier()

        def unpack(kk, j):
            for i in range(_C // 16):
                v = comb_st[pl.ds(kk * _C + i * 16, 16)]
                src_sl[j, pl.ds(i * 16, 16)] = \
                    lax.shift_right_logical(v, 14)
                dstV[j][pl.ds(i * 16, 16)] = lax.bitwise_and(v, 16383)

        def gather(kk, j):
            del kk
            idx = src_sl.at[j]

            @pl.when(c == 0)
            def _():
                pltpu.async_copy(yA_hbm.at[idx], bufV[j], gsem[j])

            @pl.when(c == 1)
            def _():
                pltpu.async_copy(yB_hbm.at[idx], bufV[j], gsem[j])

        def wait_gather(j):
            pltpu.make_async_copy(
                yA_hbm.at[src_sl.at[j]], bufV[j], gsem[j]).wait()

        def ascatter(kk, j):
            del kk
            pltpu.async_copy(bufV[j], acc_sh.at[dstV[j]], ssem[j], add=True)
            if with_deg:
                @pl.when(c == 0)
                def _():
                    pltpu.async_copy(ones_v, deg_sh.at[dstV[j]], ssem[j],
                                     add=True)

        def wait_scatter(j):
            pltpu.make_async_copy(
                bufV[j], acc_sh.at[dstV[j]], ssem[j]).wait()
            if with_deg:
                @pl.when(c == 0)
                def _():
                    pltpu.make_async_copy(
                        ones_v, deg_sh.at[dstV[j]], ssem[j]).wait()

        unpack(0, 0)
        gather(0, 0)
        unpack(1, 1)
        gather(1, 1)

        def process(kk, j, j2):
            wait_gather(j)
            ascatter(kk, j)
            wait_scatter(j2)
            unpack(kk + 2, j2)
            gather(kk + 2, j2)

        wait_gather(0)
        ascatter(0, 0)
        unpack(2, 2)
        gather(2, 2)
        wait_gather(1)
        ascatter(1, 1)
        wait_scatter(0)
        unpack(3, 0)
        gather(3, 0)

        @pl.loop(0, (_NCHUNK - 5) // 3)
        def _(i):
            k0 = 3 * i + 2
            process(k0, 2, 1)
            process(k0 + 1, 0, 2)
            process(k0 + 2, 1, 0)

        process(_NCHUNK - 3, 2, 1)
        wait_gather(0)
        ascatter(_NCHUNK - 2, 0)
        wait_gather(1)
        ascatter(_NCHUNK - 1, 1)
        wait_scatter(2)
        wait_scatter(0)
        wait_scatter(1)

        plsc.subcore_barrier()
        @pl.when(c == 0)
        def _():
            pltpu.sync_copy(acc_sh.at[pl.ds(s * _RPT, _RPT)],
                            sA_out.at[pl.ds(s * _RPT, _RPT)])

            @pl.when(s == _NS - 1)
            def _():
                pltpu.sync_copy(acc_sh.at[pl.ds(_NS * _RPT, _RTAIL)],
                                sA_out.at[pl.ds(_NS * _RPT, _RTAIL)])

        @pl.when(c == 1)
        def _():
            pltpu.sync_copy(acc_sh.at[pl.ds(s * _RPT, _RPT)],
                            sB_out.at[pl.ds(s * _RPT, _RPT)])

            @pl.when(s == _NS - 1)
            def _():
                pltpu.sync_copy(acc_sh.at[pl.ds(_NS * _RPT, _RTAIL)],
                                sB_out.at[pl.ds(_NS * _RPT, _RTAIL)])

        if with_deg:
            @pl.when(jnp.logical_and(s == 0, c == 0))
            def _():
                pltpu.sync_copy(deg_sh, deg_out)

    return k(yA, yB, comb, zrow, zdeg)


def _mm(a, b):
    return jnp.dot(a, b, preferred_element_type=jnp.float32)


def _tc_y(x, Wl):
    def body(x_ref, wl_ref, yA_ref, yB_ref):
        y = _mm(x_ref[...], wl_ref[...])
        yA_ref[...] = y[:, :_H]
        yB_ref[...] = y[:, _H:]

    return pl.pallas_call(
        body,
        grid=(_N // _BN,),
        in_specs=[
            pl.BlockSpec((_BN, _D), lambda i: (i, 0)),
            pl.BlockSpec((_D, _D), lambda i: (0, 0)),
        ],
        out_specs=[
            pl.BlockSpec((_BN, _H), lambda i: (i, 0)),
            pl.BlockSpec((_BN, _H), lambda i: (i, 0)),
        ],
        out_shape=[
            jax.ShapeDtypeStruct((_N, _H), jnp.float32),
            jax.ShapeDtypeStruct((_N, _H), jnp.float32),
        ],
    )(x, Wl)


def _tc_z(x, Wr, b):
    def body(x_ref, wr_ref, b_ref, z_ref):
        z_ref[...] = _mm(x_ref[...], wr_ref[...]) + b_ref[...]

    return pl.pallas_call(
        body,
        grid=(_N // _BN,),
        in_specs=[
            pl.BlockSpec((_BN, _D), lambda i: (i, 0)),
            pl.BlockSpec((_D, _D), lambda i: (0, 0)),
            pl.BlockSpec((1, _D), lambda i: (0, 0)),
        ],
        out_specs=pl.BlockSpec((_BN, _D), lambda i: (i, 0)),
        out_shape=jax.ShapeDtypeStruct((_N, _D), jnp.float32),
    )(x, Wr, b.reshape(1, _D))


def _tc_mid(sA, sB, deg, z, Wl):
    def body(sA_ref, sB_ref, deg_ref, z_ref, wl_ref,
             yA_ref, yB_ref, hA_ref, hB_ref):
        rd = 1.0 / jnp.maximum(deg_ref[...], 1.0)
        zb = z_ref[...]
        hA = jnp.maximum(sA_ref[...] * rd + zb[:, :_H], 0.0)
        hB = jnp.maximum(sB_ref[...] * rd + zb[:, _H:], 0.0)
        hA_ref[...] = hA
        hB_ref[...] = hB
        wl = wl_ref[...]
        y2 = _mm(hA, wl[:_H, :]) + _mm(hB, wl[_H:, :])
        yA_ref[...] = y2[:, :_H]
        yB_ref[...] = y2[:, _H:]

    return pl.pallas_call(
        body,
        grid=(_N // _BN,),
        in_specs=[
            pl.BlockSpec((_BN, _H), lambda i: (i, 0)),
            pl.BlockSpec((_BN, _H), lambda i: (i, 0)),
            pl.BlockSpec((_BN, 1), lambda i: (i, 0)),
            pl.BlockSpec((_BN, _D), lambda i: (i, 0)),
            pl.BlockSpec((_D, _D), lambda i: (0, 0)),
        ],
        out_specs=[
            pl.BlockSpec((_BN, _H), lambda i: (i, 0)),
            pl.BlockSpec((_BN, _H), lambda i: (i, 0)),
            pl.BlockSpec((_BN, _H), lambda i: (i, 0)),
            pl.BlockSpec((_BN, _H), lambda i: (i, 0)),
        ],
        out_shape=[
            jax.ShapeDtypeStruct((_N, _H), jnp.float32),
            jax.ShapeDtypeStruct((_N, _H), jnp.float32),
            jax.ShapeDtypeStruct((_N, _H), jnp.float32),
            jax.ShapeDtypeStruct((_N, _H), jnp.float32),
        ],
    )(sA, sB, deg, z, Wl)


def _tc_z2(hA, hB, Wr, b):
    def body(hA_ref, hB_ref, wr_ref, b_ref, z_ref):
        wr = wr_ref[...]
        z_ref[...] = (_mm(hA_ref[...], wr[:_H, :]) +
                      _mm(hB_ref[...], wr[_H:, :]) + b_ref[...])

    return pl.pallas_call(
        body,
        grid=(_N // _BN,),
        in_specs=[
            pl.BlockSpec((_BN, _H), lambda i: (i, 0)),
            pl.BlockSpec((_BN, _H), lambda i: (i, 0)),
            pl.BlockSpec((_D, _D), lambda i: (0, 0)),
            pl.BlockSpec((1, _D), lambda i: (0, 0)),
        ],
        out_specs=pl.BlockSpec((_BN, _D), lambda i: (i, 0)),
        out_shape=jax.ShapeDtypeStruct((_N, _D), jnp.float32),
    )(hA, hB, Wr, b.reshape(1, _D))


def _tc_post(sA, sB, deg, z):
    def body(sA_ref, sB_ref, deg_ref, z_ref, o_ref):
        rd = 1.0 / jnp.maximum(deg_ref[...], 1.0)
        o_ref[...] = jnp.concatenate(
            [sA_ref[...] * rd, sB_ref[...] * rd], axis=1) + z_ref[...]

    return pl.pallas_call(
        body,
        grid=(_N // _BN,),
        in_specs=[
            pl.BlockSpec((_BN, _H), lambda i: (i, 0)),
            pl.BlockSpec((_BN, _H), lambda i: (i, 0)),
            pl.BlockSpec((_BN, 1), lambda i: (i, 0)),
            pl.BlockSpec((_BN, _D), lambda i: (i, 0)),
        ],
        out_specs=pl.BlockSpec((_BN, _D), lambda i: (i, 0)),
        out_shape=jax.ShapeDtypeStruct((_N, _D), jnp.float32),
    )(sA, sB, deg, z)


def kernel(x, edge_index, Wl1, Wr1, b1, Wl2, Wr2, b2):
    src = edge_index[0].astype(jnp.int32)
    dst = edge_index[1].astype(jnp.int32)
    comb = (src << 14) | dst
    zrow = jnp.zeros((_NACC, _H), jnp.float32)
    zdeg = jnp.zeros((_NACC,), jnp.float32)

    yA1, yB1 = _tc_y(x, Wl1)
    s1A, s1B, deg1 = _seg_sum_sc(yA1, yB1, comb, zrow, zdeg, with_deg=True)
    z1 = _tc_z(x, Wr1, b1)
    deg = deg1[:_N].reshape(_N, 1)
    y2A, y2B, hA, hB = _tc_mid(s1A, s1B, deg, z1, Wl2)
    s2A, s2B = _seg_sum_sc(y2A, y2B, comb, zrow, zdeg, with_deg=False)
    z2 = _tc_z2(hA, hB, Wr2, b2)
    return _tc_post(s2A, s2B, deg, z2)
